# Initial kernel scaffold; baseline (speedup 1.0000x reference)
import jax
import jax.numpy as jnp
from jax.experimental import pallas as pl


def kernel(x, edge_index, W, b):
    raise NotImplementedError("write your pallas kernel here")


# speedup vs baseline: 11.7646x; 11.7646x over previous
import functools

import jax
import jax.numpy as jnp
from jax import lax
from jax.experimental import pallas as pl
from jax.experimental.pallas import tpu as pltpu
from jax.experimental.pallas import tpu_sc as plsc

N = 10000
E = 160000
D = 256
H = D // 2
NC = 2
NS = 16
CH = 128
NCHUNK = E // CH
ROWS_A = 640
ROWS_LAST = N - (NS - 1) * ROWS_A
BLK = 2000


def _copy_rows_split(src, dst, s):
    r0 = s * ROWS_A

    @pl.when(s < NS - 1)
    def _():
        pltpu.sync_copy(src.at[pl.ds(r0, ROWS_A)], dst.at[pl.ds(r0, ROWS_A)])

    @pl.when(s == NS - 1)
    def _():
        pltpu.sync_copy(src.at[pl.ds(r0, ROWS_LAST)],
                        dst.at[pl.ds(r0, ROWS_LAST)])

@functools.cache
def _sc_mesh():
    return plsc.VectorSubcoreMesh(core_axis_name="c", subcore_axis_name="s",
                                  num_cores=NC, num_subcores=NS)


def _mm_body(x_ref, w_ref, o_ref):
    o_ref[...] = jnp.dot(x_ref[...], w_ref[...],
                         preferred_element_type=jnp.float32,
                         precision=lax.Precision.HIGHEST)


def _mm(x, W):
    return pl.pallas_call(
        _mm_body,
        grid=(N // BLK,),
        in_specs=[
            pl.BlockSpec((BLK, D), lambda i: (i, 0)),
            pl.BlockSpec((D, D), lambda i: (0, 0)),
        ],
        out_specs=pl.BlockSpec((BLK, D), lambda i: (i, 0)),
        out_shape=jax.ShapeDtypeStruct((N, D), jnp.float32),
    )(x, W)


def _deg_body(dst_hbm, ones_hbm, zeros_hbm, degp_hbm, idx_v, ones_v, acc_sh):
    c = lax.axis_index("c")
    s = lax.axis_index("s")
    _copy_rows_split(zeros_hbm, acc_sh, s)
    pltpu.sync_copy(ones_hbm, ones_v)
    plsc.subcore_barrier()

    @pl.loop(NC * s + c, NCHUNK, step=NC * NS)
    def _(t):
        pltpu.sync_copy(dst_hbm.at[pl.ds(t * CH, CH)], idx_v)
        pltpu.sync_copy(ones_v, acc_sh.at[idx_v], add=True)

    plsc.subcore_barrier()
    _copy_rows_split(acc_sh, degp_hbm.at[c], s)


@functools.cache
def _deg():
    return pl.kernel(
        _deg_body,
        out_type=jax.ShapeDtypeStruct((NC, N, H), jnp.float32),
        mesh=_sc_mesh(),
        scratch_types=[
            pltpu.VMEM((CH,), jnp.int32),
            pltpu.VMEM((CH, H), jnp.float32),
            pltpu.VMEM_SHARED((N, H), jnp.float32),
        ],
    )


def _scale_body(xw_ref, degp_ref, z_ref):
    deg = degp_ref[0, :, 0:1] + degp_ref[1, :, 0:1] + 1.0
    z = xw_ref[...] * lax.rsqrt(deg)
    z_ref[0] = z[:, :H]
    z_ref[1] = z[:, H:]


def _scale(xw, degp):
    return pl.pallas_call(
        _scale_body,
        grid=(N // BLK,),
        in_specs=[
            pl.BlockSpec((BLK, D), lambda i: (i, 0)),
            pl.BlockSpec((NC, BLK, H), lambda i: (0, i, 0)),
        ],
        out_specs=pl.BlockSpec((NC, BLK, H), lambda i: (0, i, 0)),
        out_shape=jax.ShapeDtypeStruct((NC, N, H), jnp.float32),
    )(xw, degp)


def _agg_body(z_hbm, src_hbm, dst_hbm, agg_hbm, sidx, didx, rows, acc_sh, sem):
    c = lax.axis_index("c")
    s = lax.axis_index("s")
    _copy_rows_split(z_hbm.at[c], acc_sh, s)
    plsc.subcore_barrier()

    @pl.loop(s, NCHUNK, step=NS)
    def _(t):
        base = t * CH
        pltpu.sync_copy(src_hbm.at[pl.ds(base, CH)], sidx)
        pltpu.sync_copy(dst_hbm.at[pl.ds(base, CH)], didx)
        pltpu.async_copy(z_hbm.at[c].at[sidx], rows, sem).wait()
        pltpu.sync_copy(rows, acc_sh.at[didx], add=True)

    plsc.subcore_barrier()
    _copy_rows_split(acc_sh, agg_hbm.at[c], s)


@functools.cache
def _agg():
    return pl.kernel(
        _agg_body,
        out_type=jax.ShapeDtypeStruct((NC, N, H), jnp.float32),
        mesh=_sc_mesh(),
        scratch_types=[
            pltpu.VMEM((CH,), jnp.int32),
            pltpu.VMEM((CH,), jnp.int32),
            pltpu.VMEM((CH, H), jnp.float32),
            pltpu.VMEM_SHARED((N, H), jnp.float32),
            pltpu.SemaphoreType.DMA,
        ],
    )


def _fin_body(agg_ref, degp_ref, b_ref, o_ref):
    deg = degp_ref[0, :, 0:1] + degp_ref[1, :, 0:1] + 1.0
    a = jnp.concatenate([agg_ref[0], agg_ref[1]], axis=1)
    out = a * lax.rsqrt(deg) + b_ref[...]
    m = jnp.max(out, axis=1, keepdims=True)
    e = jnp.exp(out - m)
    lse = jnp.log(jnp.sum(e, axis=1, keepdims=True))
    o_ref[...] = out - m - lse


def _fin(agg, degp, b):
    return pl.pallas_call(
        _fin_body,
        grid=(N // BLK,),
        in_specs=[
            pl.BlockSpec((NC, BLK, H), lambda i: (0, i, 0)),
            pl.BlockSpec((NC, BLK, H), lambda i: (0, i, 0)),
            pl.BlockSpec((1, D), lambda i: (0, 0)),
        ],
        out_specs=pl.BlockSpec((BLK, D), lambda i: (i, 0)),
        out_shape=jax.ShapeDtypeStruct((N, D), jnp.float32),
    )(agg, degp, b)


def kernel(x, edge_index, W, b):
    src = edge_index[0].astype(jnp.int32)
    dst = edge_index[1].astype(jnp.int32)
    ones16 = jnp.ones((CH, H), jnp.float32)
    zeros16 = jnp.zeros((N, H), jnp.float32)

    xw = _mm(x, W)
    degp = _deg()(dst, ones16, zeros16)
    z = _scale(xw, degp)
    agg = _agg()(z, src, dst)
    return _fin(agg, degp, b.reshape(1, D))
